# baseline (speedup 1.0000x reference)
import functools

import jax
import jax.numpy as jnp
from jax import lax
from jax.experimental import pallas as pl
from jax.experimental.pallas import tpu as pltpu
from jax.experimental.pallas import tpu_sc as plsc

_INFO = plsc.get_sparse_core_info()
_NC, _NS = _INFO.num_cores, _INFO.num_subcores
_NW = _NC * _NS


def _transpose_call_tc(tt):
    E, V = tt.shape
    CH = 1024
    n_blocks = (V + CH - 1) // CH
    s_rows = ((V + 127) // 128) * 64

    def body(in_ref, out_ref):
        blk = in_ref[...]
        out_ref[...] = (
            blk.reshape(E, CH // 2, 2).transpose(1, 2, 0).reshape(CH // 2, 128)
        )

    return pl.pallas_call(
        body,
        grid=(n_blocks,),
        in_specs=[pl.BlockSpec((E, CH), lambda c: (0, c))],
        out_specs=pl.BlockSpec((CH // 2, 128), lambda c: (c, 0)),
        out_shape=jax.ShapeDtypeStruct((s_rows, 128), jnp.float32),
    )(tt)


def _transpose_call(tt, tail_s):
    E, V = tt.shape
    n_full = V // 128
    tail = V - n_full * 128
    extra = n_full - (n_full // _NW) * _NW
    base_cnt = n_full // _NW
    s_rows = (V + tail) // 2

    mesh = plsc.VectorSubcoreMesh(core_axis_name="c", subcore_axis_name="s")

    @functools.partial(
        pl.kernel,
        mesh=mesh,
        out_type=jax.ShapeDtypeStruct((s_rows, 128), jnp.float32),
        scratch_types=[
            pltpu.VMEM((E, 128), jnp.float32),
            pltpu.VMEM((64, 128), jnp.float32),
        ],
        compiler_params=pltpu.CompilerParams(
            use_tc_tiling_on_sc=True, needs_layout_passes=False),
    )
    def tr_kernel(tt_hbm, tail_hbm, s_hbm, in_v, out_v):
        wid = lax.axis_index("s") * _NC + lax.axis_index("c")
        start = wid * base_cnt + jnp.minimum(wid, extra)
        cnt = base_cnt + jnp.where(wid < extra, 1, 0)

        row16 = lax.iota(jnp.int32, 16)

        def do_chunk(col_off):
            col_off = pl.multiple_of(col_off, 128)
            pltpu.sync_copy(tt_hbm.at[:, pl.ds(col_off, 128)], in_v)

            def row_body(r, carry):
                for g in range(8):
                    rvec = row16 + 16 * (g % 4)
                    cvec = jnp.full((16,), 2 * r + (1 if g >= 4 else 0),
                                    jnp.int32)
                    vals = plsc.load_gather(in_v, [rvec, cvec])
                    out_v[r, pl.ds(16 * g, 16)] = vals
                return carry

            lax.fori_loop(0, 64, row_body, 0, unroll=4)
            row_off = pl.multiple_of(col_off // 2, 64)
            pltpu.sync_copy(out_v, s_hbm.at[pl.ds(row_off, 64), :])

        def loop_body(c, carry):
            do_chunk((start + c) * 128)
            return carry

        lax.fori_loop(0, cnt, loop_body, 0)

        @pl.when(wid == _NW - 1)
        def _tail():
            pltpu.sync_copy(tail_hbm, out_v.at[pl.ds(0, 32), :])
            pltpu.sync_copy(out_v.at[pl.ds(0, 32), :],
                            s_hbm.at[pl.ds(n_full * 64, 32), :])

    return tr_kernel(tt, tail_s)


def _gather_call(idx_flat, table_rm, N, D):
    n_per_w = N // _NW
    CH = 800
    NB = 2
    n_ch = n_per_w // CH

    mesh = plsc.VectorSubcoreMesh(core_axis_name="c", subcore_axis_name="s")

    @functools.partial(
        pl.kernel,
        mesh=mesh,
        out_type=jax.ShapeDtypeStruct((N, D), jnp.float32),
        scratch_types=[
            pltpu.VMEM((n_per_w,), jnp.int32),
            [pltpu.VMEM((CH, D), jnp.float32) for _ in range(NB)],
            [pltpu.SemaphoreType.DMA for _ in range(NB)],
            [pltpu.SemaphoreType.DMA for _ in range(NB)],
        ],
        compiler_params=pltpu.CompilerParams(use_tc_tiling_on_sc=False),
    )
    def gather_kernel(idx_hbm, table_hbm, out_hbm, idx_v, bufs, gsems, ssems):
        wid = lax.axis_index("s") * _NC + lax.axis_index("c")
        base = wid * n_per_w

        pltpu.sync_copy(idx_hbm.at[pl.ds(base, n_per_w)], idx_v)

        def start_g(i):
            return pltpu.async_copy(
                table_hbm.at[idx_v.at[pl.ds(i * CH, CH)]], bufs[i % NB],
                gsems[i % NB])

        def start_s(i):
            return pltpu.async_copy(
                bufs[i % NB], out_hbm.at[pl.ds(base + i * CH, CH)],
                ssems[i % NB])

        gcopies = [None] * n_ch
        scopies = [None] * n_ch
        for i in range(min(NB, n_ch)):
            gcopies[i] = start_g(i)
        for i in range(n_ch):
            gcopies[i].wait()
            scopies[i] = start_s(i)
            if i + NB < n_ch:
                scopies[i].wait()
                gcopies[i + NB] = start_g(i + NB)
        for i in range(max(0, n_ch - NB), n_ch):
            scopies[i].wait()

    return gather_kernel(idx_flat, table_rm)


def kernel(x, table):
    B, L = x.shape
    V, D = table.shape
    N = B * L

    tt = jnp.swapaxes(table, 0, 1)
    s = _transpose_call_tc(tt)
    table_rm = jnp.reshape(s, (2 * s.shape[0], D))
    idx_flat = x.reshape(N)

    out = _gather_call(idx_flat, table_rm, N, D)
    return out.reshape(B, L, D)


# speedup vs baseline: 14.2645x; 14.2645x over previous
import functools

import jax
import jax.numpy as jnp
from jax import lax
from jax.experimental import pallas as pl
from jax.experimental.pallas import tpu as pltpu
from jax.experimental.pallas import tpu_sc as plsc

_INFO = plsc.get_sparse_core_info()
_NC, _NS = _INFO.num_cores, _INFO.num_subcores
_NW = _NC * _NS


def _transpose_call_tc(tt):
    E, V = tt.shape
    W = 1024
    nb = -(-V // (2 * W))
    H = W * nb
    last_blk = (V + W - 1) // W - 1

    def body(a_ref, b_ref, out_ref):
        out_ref[:, 0:E] = a_ref[...].T
        out_ref[:, E:2 * E] = b_ref[...].T

    s = pl.pallas_call(
        body,
        grid=(nb,),
        in_specs=[pl.BlockSpec((E, W), lambda c: (0, c)),
                  pl.BlockSpec((E, W),
                               lambda c: (0, jnp.minimum(c + nb, last_blk)))],
        out_specs=pl.BlockSpec((W, 2 * E), lambda c: (c, 0)),
        out_shape=jax.ShapeDtypeStruct((H, 2 * E), jnp.float32),
    )(tt, tt)
    return s, H


def _transpose_call(tt, tail_s):
    E, V = tt.shape
    n_full = V // 128
    tail = V - n_full * 128
    extra = n_full - (n_full // _NW) * _NW
    base_cnt = n_full // _NW
    s_rows = (V + tail) // 2

    mesh = plsc.VectorSubcoreMesh(core_axis_name="c", subcore_axis_name="s")

    @functools.partial(
        pl.kernel,
        mesh=mesh,
        out_type=jax.ShapeDtypeStruct((s_rows, 128), jnp.float32),
        scratch_types=[
            pltpu.VMEM((E, 128), jnp.float32),
            pltpu.VMEM((64, 128), jnp.float32),
        ],
        compiler_params=pltpu.CompilerParams(
            use_tc_tiling_on_sc=True, needs_layout_passes=False),
    )
    def tr_kernel(tt_hbm, tail_hbm, s_hbm, in_v, out_v):
        wid = lax.axis_index("s") * _NC + lax.axis_index("c")
        start = wid * base_cnt + jnp.minimum(wid, extra)
        cnt = base_cnt + jnp.where(wid < extra, 1, 0)

        row16 = lax.iota(jnp.int32, 16)

        def do_chunk(col_off):
            col_off = pl.multiple_of(col_off, 128)
            pltpu.sync_copy(tt_hbm.at[:, pl.ds(col_off, 128)], in_v)

            def row_body(r, carry):
                for g in range(8):
                    rvec = row16 + 16 * (g % 4)
                    cvec = jnp.full((16,), 2 * r + (1 if g >= 4 else 0),
                                    jnp.int32)
                    vals = plsc.load_gather(in_v, [rvec, cvec])
                    out_v[r, pl.ds(16 * g, 16)] = vals
                return carry

            lax.fori_loop(0, 64, row_body, 0, unroll=4)
            row_off = pl.multiple_of(col_off // 2, 64)
            pltpu.sync_copy(out_v, s_hbm.at[pl.ds(row_off, 64), :])

        def loop_body(c, carry):
            do_chunk((start + c) * 128)
            return carry

        lax.fori_loop(0, cnt, loop_body, 0)

        @pl.when(wid == _NW - 1)
        def _tail():
            pltpu.sync_copy(tail_hbm, out_v.at[pl.ds(0, 32), :])
            pltpu.sync_copy(out_v.at[pl.ds(0, 32), :],
                            s_hbm.at[pl.ds(n_full * 64, 32), :])

    return tr_kernel(tt, tail_s)


def _gather_call(idx_flat, table_rm, N, D):
    n_per_w = N // _NW
    CH = 800
    NB = 2
    n_ch = n_per_w // CH

    mesh = plsc.VectorSubcoreMesh(core_axis_name="c", subcore_axis_name="s")

    @functools.partial(
        pl.kernel,
        mesh=mesh,
        out_type=jax.ShapeDtypeStruct((N, D), jnp.float32),
        scratch_types=[
            pltpu.VMEM((n_per_w,), jnp.int32),
            [pltpu.VMEM((CH, D), jnp.float32) for _ in range(NB)],
            [pltpu.SemaphoreType.DMA for _ in range(NB)],
            [pltpu.SemaphoreType.DMA for _ in range(NB)],
        ],
        compiler_params=pltpu.CompilerParams(use_tc_tiling_on_sc=False),
    )
    def gather_kernel(idx_hbm, table_hbm, out_hbm, idx_v, bufs, gsems, ssems):
        wid = lax.axis_index("s") * _NC + lax.axis_index("c")
        base = wid * n_per_w

        pltpu.sync_copy(idx_hbm.at[pl.ds(base, n_per_w)], idx_v)

        def start_g(i):
            return pltpu.async_copy(
                table_hbm.at[idx_v.at[pl.ds(i * CH, CH)]], bufs[i % NB],
                gsems[i % NB])

        def start_s(i):
            return pltpu.async_copy(
                bufs[i % NB], out_hbm.at[pl.ds(base + i * CH, CH)],
                ssems[i % NB])

        gcopies = [None] * n_ch
        scopies = [None] * n_ch
        for i in range(min(NB, n_ch)):
            gcopies[i] = start_g(i)
        for i in range(n_ch):
            gcopies[i].wait()
            scopies[i] = start_s(i)
            if i + NB < n_ch:
                scopies[i].wait()
                gcopies[i + NB] = start_g(i + NB)
        for i in range(max(0, n_ch - NB), n_ch):
            scopies[i].wait()

    return gather_kernel(idx_flat, table_rm)


def kernel(x, table):
    B, L = x.shape
    V, D = table.shape
    N = B * L

    tt = jnp.swapaxes(table, 0, 1)
    s, H = _transpose_call_tc(tt)
    table_rm = jnp.reshape(s, (2 * H, D))
    xf = x.reshape(N)
    idx_flat = 2 * (xf % H) + xf // H

    out = _gather_call(idx_flat, table_rm, N, D)
    return out.reshape(B, L, D)


# speedup vs baseline: 20.1756x; 1.4144x over previous
import functools

import jax
import jax.numpy as jnp
from jax import lax
from jax.experimental import pallas as pl
from jax.experimental.pallas import tpu as pltpu
from jax.experimental.pallas import tpu_sc as plsc

_INFO = plsc.get_sparse_core_info()
_NC, _NS = _INFO.num_cores, _INFO.num_subcores
_NW = _NC * _NS


def _transpose_call_tc(tt):
    E, V = tt.shape
    W = 4096
    nb = -(-V // (2 * W))
    H = W * nb
    last_blk = (V + W - 1) // W - 1

    def body(a_ref, b_ref, eye_ref, out_ref):
        out_ref[:, 0:E] = a_ref[...].T
        out_ref[:, E:2 * E] = b_ref[...].T

    s = pl.pallas_call(
        body,
        grid=(nb,),
        in_specs=[pl.BlockSpec((E, W), lambda c: (0, c)),
                  pl.BlockSpec((E, W),
                               lambda c: (0, jnp.minimum(c + nb, last_blk))),
                  pl.BlockSpec((E, E), lambda c: (0, 0))],
        out_specs=pl.BlockSpec((W, 2 * E), lambda c: (c, 0)),
        out_shape=jax.ShapeDtypeStruct((H, 2 * E), jnp.float32),
    )(tt, tt, jnp.eye(E, dtype=jnp.float32))
    return s, H


def _transpose_call(tt, tail_s):
    E, V = tt.shape
    n_full = V // 128
    tail = V - n_full * 128
    extra = n_full - (n_full // _NW) * _NW
    base_cnt = n_full // _NW
    s_rows = (V + tail) // 2

    mesh = plsc.VectorSubcoreMesh(core_axis_name="c", subcore_axis_name="s")

    @functools.partial(
        pl.kernel,
        mesh=mesh,
        out_type=jax.ShapeDtypeStruct((s_rows, 128), jnp.float32),
        scratch_types=[
            pltpu.VMEM((E, 128), jnp.float32),
            pltpu.VMEM((64, 128), jnp.float32),
        ],
        compiler_params=pltpu.CompilerParams(
            use_tc_tiling_on_sc=True, needs_layout_passes=False),
    )
    def tr_kernel(tt_hbm, tail_hbm, s_hbm, in_v, out_v):
        wid = lax.axis_index("s") * _NC + lax.axis_index("c")
        start = wid * base_cnt + jnp.minimum(wid, extra)
        cnt = base_cnt + jnp.where(wid < extra, 1, 0)

        row16 = lax.iota(jnp.int32, 16)

        def do_chunk(col_off):
            col_off = pl.multiple_of(col_off, 128)
            pltpu.sync_copy(tt_hbm.at[:, pl.ds(col_off, 128)], in_v)

            def row_body(r, carry):
                for g in range(8):
                    rvec = row16 + 16 * (g % 4)
                    cvec = jnp.full((16,), 2 * r + (1 if g >= 4 else 0),
                                    jnp.int32)
                    vals = plsc.load_gather(in_v, [rvec, cvec])
                    out_v[r, pl.ds(16 * g, 16)] = vals
                return carry

            lax.fori_loop(0, 64, row_body, 0, unroll=4)
            row_off = pl.multiple_of(col_off // 2, 64)
            pltpu.sync_copy(out_v, s_hbm.at[pl.ds(row_off, 64), :])

        def loop_body(c, carry):
            do_chunk((start + c) * 128)
            return carry

        lax.fori_loop(0, cnt, loop_body, 0)

        @pl.when(wid == _NW - 1)
        def _tail():
            pltpu.sync_copy(tail_hbm, out_v.at[pl.ds(0, 32), :])
            pltpu.sync_copy(out_v.at[pl.ds(0, 32), :],
                            s_hbm.at[pl.ds(n_full * 64, 32), :])

    return tr_kernel(tt, tail_s)


def _gather_call(idx_flat, table_rm, N, D):
    n_per_w = N // _NW
    CH = 800
    NB = 2
    n_ch = n_per_w // CH

    mesh = plsc.VectorSubcoreMesh(core_axis_name="c", subcore_axis_name="s")

    @functools.partial(
        pl.kernel,
        mesh=mesh,
        out_type=jax.ShapeDtypeStruct((N, D), jnp.float32),
        scratch_types=[
            pltpu.VMEM((n_per_w,), jnp.int32),
            [pltpu.VMEM((CH, D), jnp.float32) for _ in range(NB)],
            [pltpu.SemaphoreType.DMA for _ in range(NB)],
            [pltpu.SemaphoreType.DMA for _ in range(NB)],
        ],
        compiler_params=pltpu.CompilerParams(use_tc_tiling_on_sc=False),
    )
    def gather_kernel(idx_hbm, table_hbm, out_hbm, idx_v, bufs, gsems, ssems):
        wid = lax.axis_index("s") * _NC + lax.axis_index("c")
        base = wid * n_per_w

        pltpu.sync_copy(idx_hbm.at[pl.ds(base, n_per_w)], idx_v)

        def start_g(i):
            return pltpu.async_copy(
                table_hbm.at[idx_v.at[pl.ds(i * CH, CH)]], bufs[i % NB],
                gsems[i % NB])

        def start_s(i):
            return pltpu.async_copy(
                bufs[i % NB], out_hbm.at[pl.ds(base + i * CH, CH)],
                ssems[i % NB])

        gcopies = [None] * n_ch
        scopies = [None] * n_ch
        for i in range(min(NB, n_ch)):
            gcopies[i] = start_g(i)
        for i in range(n_ch):
            gcopies[i].wait()
            scopies[i] = start_s(i)
            if i + NB < n_ch:
                scopies[i].wait()
                gcopies[i + NB] = start_g(i + NB)
        for i in range(max(0, n_ch - NB), n_ch):
            scopies[i].wait()

    return gather_kernel(idx_flat, table_rm)


def kernel(x, table):
    B, L = x.shape
    V, D = table.shape
    N = B * L

    tt = jnp.swapaxes(table, 0, 1)
    s, H = _transpose_call_tc(tt)
    table_rm = jnp.reshape(s, (2 * H, D))
    xf = x.reshape(N)
    idx_flat = 2 * (xf % H) + xf // H

    out = _gather_call(idx_flat, table_rm, N, D)
    return out.reshape(B, L, D)


# speedup vs baseline: 21.8056x; 1.0808x over previous
import functools

import jax
import jax.numpy as jnp
from jax import lax
from jax.experimental import pallas as pl
from jax.experimental.pallas import tpu as pltpu
from jax.experimental.pallas import tpu_sc as plsc

_INFO = plsc.get_sparse_core_info()
_NC, _NS = _INFO.num_cores, _INFO.num_subcores
_NW = _NC * _NS


def _transpose_call_tc(tt):
    E, V = tt.shape
    W = 8192
    nb = -(-V // (2 * W))
    H = W * nb
    last_blk = (V + W - 1) // W - 1

    def body(a_ref, b_ref, eye_ref, out_ref):
        out_ref[:, 0:E] = a_ref[...].T
        out_ref[:, E:2 * E] = b_ref[...].T

    s = pl.pallas_call(
        body,
        grid=(nb,),
        in_specs=[pl.BlockSpec((E, W), lambda c: (0, c)),
                  pl.BlockSpec((E, W),
                               lambda c: (0, jnp.minimum(c + nb, last_blk))),
                  pl.BlockSpec((E, E), lambda c: (0, 0))],
        out_specs=pl.BlockSpec((W, 2 * E), lambda c: (c, 0)),
        out_shape=jax.ShapeDtypeStruct((H, 2 * E), jnp.float32),
    )(tt, tt, jnp.eye(E, dtype=jnp.float32))
    return s, H


def _transpose_call(tt, tail_s):
    E, V = tt.shape
    n_full = V // 128
    tail = V - n_full * 128
    extra = n_full - (n_full // _NW) * _NW
    base_cnt = n_full // _NW
    s_rows = (V + tail) // 2

    mesh = plsc.VectorSubcoreMesh(core_axis_name="c", subcore_axis_name="s")

    @functools.partial(
        pl.kernel,
        mesh=mesh,
        out_type=jax.ShapeDtypeStruct((s_rows, 128), jnp.float32),
        scratch_types=[
            pltpu.VMEM((E, 128), jnp.float32),
            pltpu.VMEM((64, 128), jnp.float32),
        ],
        compiler_params=pltpu.CompilerParams(
            use_tc_tiling_on_sc=True, needs_layout_passes=False),
    )
    def tr_kernel(tt_hbm, tail_hbm, s_hbm, in_v, out_v):
        wid = lax.axis_index("s") * _NC + lax.axis_index("c")
        start = wid * base_cnt + jnp.minimum(wid, extra)
        cnt = base_cnt + jnp.where(wid < extra, 1, 0)

        row16 = lax.iota(jnp.int32, 16)

        def do_chunk(col_off):
            col_off = pl.multiple_of(col_off, 128)
            pltpu.sync_copy(tt_hbm.at[:, pl.ds(col_off, 128)], in_v)

            def row_body(r, carry):
                for g in range(8):
                    rvec = row16 + 16 * (g % 4)
                    cvec = jnp.full((16,), 2 * r + (1 if g >= 4 else 0),
                                    jnp.int32)
                    vals = plsc.load_gather(in_v, [rvec, cvec])
                    out_v[r, pl.ds(16 * g, 16)] = vals
                return carry

            lax.fori_loop(0, 64, row_body, 0, unroll=4)
            row_off = pl.multiple_of(col_off // 2, 64)
            pltpu.sync_copy(out_v, s_hbm.at[pl.ds(row_off, 64), :])

        def loop_body(c, carry):
            do_chunk((start + c) * 128)
            return carry

        lax.fori_loop(0, cnt, loop_body, 0)

        @pl.when(wid == _NW - 1)
        def _tail():
            pltpu.sync_copy(tail_hbm, out_v.at[pl.ds(0, 32), :])
            pltpu.sync_copy(out_v.at[pl.ds(0, 32), :],
                            s_hbm.at[pl.ds(n_full * 64, 32), :])

    return tr_kernel(tt, tail_s)


def _gather_call(idx_flat, table_rm, N, D):
    n_per_w = N // _NW
    CH = 800
    NB = 2
    n_ch = n_per_w // CH

    mesh = plsc.VectorSubcoreMesh(core_axis_name="c", subcore_axis_name="s")

    @functools.partial(
        pl.kernel,
        mesh=mesh,
        out_type=jax.ShapeDtypeStruct((N, D), jnp.float32),
        scratch_types=[
            pltpu.VMEM((n_per_w,), jnp.int32),
            [pltpu.VMEM((CH, D), jnp.float32) for _ in range(NB)],
            [pltpu.SemaphoreType.DMA for _ in range(NB)],
            [pltpu.SemaphoreType.DMA for _ in range(NB)],
        ],
        compiler_params=pltpu.CompilerParams(use_tc_tiling_on_sc=False),
    )
    def gather_kernel(idx_hbm, table_hbm, out_hbm, idx_v, bufs, gsems, ssems):
        wid = lax.axis_index("s") * _NC + lax.axis_index("c")
        base = wid * n_per_w

        pltpu.sync_copy(idx_hbm.at[pl.ds(base, n_per_w)], idx_v)

        def start_g(i):
            return pltpu.async_copy(
                table_hbm.at[idx_v.at[pl.ds(i * CH, CH)]], bufs[i % NB],
                gsems[i % NB])

        def start_s(i):
            return pltpu.async_copy(
                bufs[i % NB], out_hbm.at[pl.ds(base + i * CH, CH)],
                ssems[i % NB])

        gcopies = [None] * n_ch
        scopies = [None] * n_ch
        for i in range(min(NB, n_ch)):
            gcopies[i] = start_g(i)
        for i in range(n_ch):
            gcopies[i].wait()
            scopies[i] = start_s(i)
            if i + NB < n_ch:
                scopies[i].wait()
                gcopies[i + NB] = start_g(i + NB)
        for i in range(max(0, n_ch - NB), n_ch):
            scopies[i].wait()

    return gather_kernel(idx_flat, table_rm)


def kernel(x, table):
    B, L = x.shape
    V, D = table.shape
    N = B * L

    tt = jnp.swapaxes(table, 0, 1)
    s, H = _transpose_call_tc(tt)
    table_rm = jnp.reshape(s, (2 * H, D))
    xf = x.reshape(N)
    idx_flat = 2 * (xf % H) + xf // H

    out = _gather_call(idx_flat, table_rm, N, D)
    return out.reshape(B, L, D)


# speedup vs baseline: 22.5780x; 1.0354x over previous
import functools

import jax
import jax.numpy as jnp
from jax import lax
from jax.experimental import pallas as pl
from jax.experimental.pallas import tpu as pltpu
from jax.experimental.pallas import tpu_sc as plsc

_INFO = plsc.get_sparse_core_info()
_NC, _NS = _INFO.num_cores, _INFO.num_subcores
_NW = _NC * _NS


def _transpose_call_tc(tt):
    E, V = tt.shape
    W = 16384
    nb = -(-V // (2 * W))
    H = W * nb
    last_blk = (V + W - 1) // W - 1

    def body(a_ref, b_ref, eye_ref, out_ref):
        out_ref[:, 0:E] = a_ref[...].T
        out_ref[:, E:2 * E] = b_ref[...].T

    s = pl.pallas_call(
        body,
        grid=(nb,),
        in_specs=[pl.BlockSpec((E, W), lambda c: (0, c)),
                  pl.BlockSpec((E, W),
                               lambda c: (0, jnp.minimum(c + nb, last_blk))),
                  pl.BlockSpec((E, E), lambda c: (0, 0))],
        out_specs=pl.BlockSpec((W, 2 * E), lambda c: (c, 0)),
        out_shape=jax.ShapeDtypeStruct((H, 2 * E), jnp.float32),
    )(tt, tt, jnp.eye(E, dtype=jnp.float32))
    return s, H


def _transpose_call(tt, tail_s):
    E, V = tt.shape
    n_full = V // 128
    tail = V - n_full * 128
    extra = n_full - (n_full // _NW) * _NW
    base_cnt = n_full // _NW
    s_rows = (V + tail) // 2

    mesh = plsc.VectorSubcoreMesh(core_axis_name="c", subcore_axis_name="s")

    @functools.partial(
        pl.kernel,
        mesh=mesh,
        out_type=jax.ShapeDtypeStruct((s_rows, 128), jnp.float32),
        scratch_types=[
            pltpu.VMEM((E, 128), jnp.float32),
            pltpu.VMEM((64, 128), jnp.float32),
        ],
        compiler_params=pltpu.CompilerParams(
            use_tc_tiling_on_sc=True, needs_layout_passes=False),
    )
    def tr_kernel(tt_hbm, tail_hbm, s_hbm, in_v, out_v):
        wid = lax.axis_index("s") * _NC + lax.axis_index("c")
        start = wid * base_cnt + jnp.minimum(wid, extra)
        cnt = base_cnt + jnp.where(wid < extra, 1, 0)

        row16 = lax.iota(jnp.int32, 16)

        def do_chunk(col_off):
            col_off = pl.multiple_of(col_off, 128)
            pltpu.sync_copy(tt_hbm.at[:, pl.ds(col_off, 128)], in_v)

            def row_body(r, carry):
                for g in range(8):
                    rvec = row16 + 16 * (g % 4)
                    cvec = jnp.full((16,), 2 * r + (1 if g >= 4 else 0),
                                    jnp.int32)
                    vals = plsc.load_gather(in_v, [rvec, cvec])
                    out_v[r, pl.ds(16 * g, 16)] = vals
                return carry

            lax.fori_loop(0, 64, row_body, 0, unroll=4)
            row_off = pl.multiple_of(col_off // 2, 64)
            pltpu.sync_copy(out_v, s_hbm.at[pl.ds(row_off, 64), :])

        def loop_body(c, carry):
            do_chunk((start + c) * 128)
            return carry

        lax.fori_loop(0, cnt, loop_body, 0)

        @pl.when(wid == _NW - 1)
        def _tail():
            pltpu.sync_copy(tail_hbm, out_v.at[pl.ds(0, 32), :])
            pltpu.sync_copy(out_v.at[pl.ds(0, 32), :],
                            s_hbm.at[pl.ds(n_full * 64, 32), :])

    return tr_kernel(tt, tail_s)


def _gather_call(idx_flat, table_rm, N, D):
    n_per_w = N // _NW
    CH = 800
    NB = 2
    n_ch = n_per_w // CH

    mesh = plsc.VectorSubcoreMesh(core_axis_name="c", subcore_axis_name="s")

    @functools.partial(
        pl.kernel,
        mesh=mesh,
        out_type=jax.ShapeDtypeStruct((N, D), jnp.float32),
        scratch_types=[
            pltpu.VMEM((n_per_w,), jnp.int32),
            [pltpu.VMEM((CH, D), jnp.float32) for _ in range(NB)],
            [pltpu.SemaphoreType.DMA for _ in range(NB)],
            [pltpu.SemaphoreType.DMA for _ in range(NB)],
        ],
        compiler_params=pltpu.CompilerParams(use_tc_tiling_on_sc=False),
    )
    def gather_kernel(idx_hbm, table_hbm, out_hbm, idx_v, bufs, gsems, ssems):
        wid = lax.axis_index("s") * _NC + lax.axis_index("c")
        base = wid * n_per_w

        pltpu.sync_copy(idx_hbm.at[pl.ds(base, n_per_w)], idx_v)

        def start_g(i):
            return pltpu.async_copy(
                table_hbm.at[idx_v.at[pl.ds(i * CH, CH)]], bufs[i % NB],
                gsems[i % NB])

        def start_s(i):
            return pltpu.async_copy(
                bufs[i % NB], out_hbm.at[pl.ds(base + i * CH, CH)],
                ssems[i % NB])

        gcopies = [None] * n_ch
        scopies = [None] * n_ch
        for i in range(min(NB, n_ch)):
            gcopies[i] = start_g(i)
        for i in range(n_ch):
            gcopies[i].wait()
            scopies[i] = start_s(i)
            if i + NB < n_ch:
                scopies[i].wait()
                gcopies[i + NB] = start_g(i + NB)
        for i in range(max(0, n_ch - NB), n_ch):
            scopies[i].wait()

    return gather_kernel(idx_flat, table_rm)


def kernel(x, table):
    B, L = x.shape
    V, D = table.shape
    N = B * L

    tt = jnp.swapaxes(table, 0, 1)
    s, H = _transpose_call_tc(tt)
    table_rm = jnp.reshape(s, (2 * H, D))
    xf = x.reshape(N)
    idx_flat = 2 * (xf % H) + xf // H

    out = _gather_call(idx_flat, table_rm, N, D)
    return out.reshape(B, L, D)


# speedup vs baseline: 22.6081x; 1.0013x over previous
import functools

import jax
import jax.numpy as jnp
from jax import lax
from jax.experimental import pallas as pl
from jax.experimental.pallas import tpu as pltpu
from jax.experimental.pallas import tpu_sc as plsc

_INFO = plsc.get_sparse_core_info()
_NC, _NS = _INFO.num_cores, _INFO.num_subcores
_NW = _NC * _NS


def _transpose_call_tc(tt):
    E, V = tt.shape
    W = 16384
    nb = -(-V // (2 * W))
    H = W * nb
    last_blk = (V + W - 1) // W - 1

    def body(a_ref, b_ref, out_ref):
        out_ref[:, 0:E] = a_ref[...].T
        out_ref[:, E:2 * E] = b_ref[...].T

    s = pl.pallas_call(
        body,
        grid=(nb,),
        in_specs=[pl.BlockSpec((E, W), lambda c: (0, c)),
                  pl.BlockSpec((E, W),
                               lambda c: (0, jnp.minimum(c + nb, last_blk)))],
        out_specs=pl.BlockSpec((W, 2 * E), lambda c: (c, 0)),
        out_shape=jax.ShapeDtypeStruct((H, 2 * E), jnp.float32),
    )(tt, tt)
    return s, H


def _gather_call(idx_flat, table_rm, N, D):
    n_per_w = N // _NW
    CH = 800
    NB = 2
    n_ch = n_per_w // CH

    mesh = plsc.VectorSubcoreMesh(core_axis_name="c", subcore_axis_name="s")

    @functools.partial(
        pl.kernel,
        mesh=mesh,
        out_type=jax.ShapeDtypeStruct((N, D), jnp.float32),
        scratch_types=[
            pltpu.VMEM((n_per_w,), jnp.int32),
            [pltpu.VMEM((CH, D), jnp.float32) for _ in range(NB)],
            [pltpu.SemaphoreType.DMA for _ in range(NB)],
            [pltpu.SemaphoreType.DMA for _ in range(NB)],
        ],
        compiler_params=pltpu.CompilerParams(use_tc_tiling_on_sc=False),
    )
    def gather_kernel(idx_hbm, table_hbm, out_hbm, idx_v, bufs, gsems, ssems):
        wid = lax.axis_index("s") * _NC + lax.axis_index("c")
        base = wid * n_per_w

        pltpu.sync_copy(idx_hbm.at[pl.ds(base, n_per_w)], idx_v)

        def start_g(i):
            return pltpu.async_copy(
                table_hbm.at[idx_v.at[pl.ds(i * CH, CH)]], bufs[i % NB],
                gsems[i % NB])

        def start_s(i):
            return pltpu.async_copy(
                bufs[i % NB], out_hbm.at[pl.ds(base + i * CH, CH)],
                ssems[i % NB])

        gcopies = [None] * n_ch
        scopies = [None] * n_ch
        for i in range(min(NB, n_ch)):
            gcopies[i] = start_g(i)
        for i in range(n_ch):
            gcopies[i].wait()
            scopies[i] = start_s(i)
            if i + NB < n_ch:
                scopies[i].wait()
                gcopies[i + NB] = start_g(i + NB)
        for i in range(max(0, n_ch - NB), n_ch):
            scopies[i].wait()

    return gather_kernel(idx_flat, table_rm)


def kernel(x, table):
    B, L = x.shape
    V, D = table.shape
    N = B * L

    tt = jnp.swapaxes(table, 0, 1)
    s, H = _transpose_call_tc(tt)
    table_rm = jnp.reshape(s, (2 * H, D))
    xf = x.reshape(N)
    idx_flat = 2 * (xf % H) + xf // H

    out = _gather_call(idx_flat, table_rm, N, D)
    return out.reshape(B, L, D)
